# baseline (speedup 1.0000x reference)
import functools

import jax
import jax.numpy as jnp
from jax import lax
from jax.experimental import pallas as pl
from jax.experimental.pallas import tpu as pltpu
from jax.experimental.pallas import tpu_sc as plsc

B = 1024
N = 100000
D = 128
H = 512

TN = 4096
GRID = (N + TN - 1) // TN

_NC = 2
_NS = 16
_NW = _NC * _NS
_BPW = B // _NW


_RT = 64
_NS_STRIPS = TN // 128
_BIG = 3.4e38


_NSTEP = (GRID + 1) // 2 + 1


def _argmin_body(codes_ref, tab_ref, out_ref, runm_ref, runi_ref,
                 s2a_ref, s2b_ref, c2a_ref, c2b_ref):
    step = pl.program_id(0)

    @pl.when(step == 0)
    def _init():
        runm_ref[...] = jnp.full((B, 128), _BIG, jnp.float32)
        s2b_ref[...] = jnp.zeros((B, TN), jnp.float32)
        c2b_ref[...] = jnp.full((1, TN), _BIG, jnp.float32)

    codes = codes_ref[...]
    q2 = jnp.sum(codes * codes, axis=1, keepdims=True)
    codes2 = codes + codes
    col = lax.broadcasted_iota(jnp.int32, (1, TN), 1)

    def produce(blk, tab, s2_ref, c2_ref):
        c2 = jnp.sum(tab * tab, axis=1)[None, :]
        c2_ref[...] = jnp.where(col < N - blk * TN, c2, _BIG)
        s2_ref[...] = lax.dot_general(
            codes2, tab, (((1,), (1,)), ((), ())),
            preferred_element_type=jnp.float32)

    def consume(blk, s2_ref, c2_ref):
        c2 = c2_ref[...]
        for rt in range(B // _RT):
            rsl = slice(rt * _RT, (rt + 1) * _RT)
            q2t = q2[rsl, :]
            rm = runm_ref[rsl, :]
            ri = runi_ref[rsl, :]
            for k in range(_NS_STRIPS):
                csl = slice(k * 128, (k + 1) * 128)
                d2k = (q2t + c2[:, csl]) - s2_ref[rsl, csl]
                ck = d2k < rm
                rm = jnp.where(ck, d2k, rm)
                ri = jnp.where(ck, blk * _NS_STRIPS + jnp.int32(k), ri)
            runm_ref[rsl, :] = rm
            runi_ref[rsl, :] = ri

    tab2 = tab_ref[...]
    produce(2 * step, tab2[:TN, :], s2a_ref, c2a_ref)
    consume(2 * step - 1, s2b_ref, c2b_ref)
    produce(2 * step + 1, tab2[TN:, :], s2b_ref, c2b_ref)
    consume(2 * step, s2a_ref, c2a_ref)

    @pl.when(step == _NSTEP - 1)
    def _tail():
        for rt in range(B // _RT):
            rsl = slice(rt * _RT, (rt + 1) * _RT)
            rm = runm_ref[rsl, :]
            ri = runi_ref[rsl, :]
            m = jnp.min(rm, axis=1, keepdims=True)
            gi = ri * 128 + lax.broadcasted_iota(jnp.int32, (_RT, 128), 1)
            am = jnp.min(jnp.where(rm == m, gi, jnp.int32(2**31 - 1)),
                         axis=1, keepdims=True)
            out_ref[rsl, :] = am


def _argmin_call(codes, codes_table):
    nblk2 = (N + 2 * TN - 1) // (2 * TN)
    return pl.pallas_call(
        _argmin_body,
        grid=(_NSTEP,),
        in_specs=[
            pl.BlockSpec((B, D), lambda i: (0, 0)),
            pl.BlockSpec((2 * TN, D), lambda i: (jnp.minimum(i, nblk2 - 1), 0)),
        ],
        out_specs=pl.BlockSpec((B, 1), lambda i: (0, 0)),
        out_shape=jax.ShapeDtypeStruct((B, 1), jnp.int32),
        scratch_shapes=[
            pltpu.VMEM((B, 128), jnp.float32),
            pltpu.VMEM((B, 128), jnp.int32),
            pltpu.VMEM((B, TN), jnp.float32),
            pltpu.VMEM((B, TN), jnp.float32),
            pltpu.VMEM((1, TN), jnp.float32),
            pltpu.VMEM((1, TN), jnp.float32),
        ],
        compiler_params=pltpu.CompilerParams(
            dimension_semantics=("arbitrary",),
        ),
    )(codes, codes_table)


def _gather_body(tab_hbm, idx_hbm, out_hbm, idx_v, rows_v, sem):
    wid = lax.axis_index("s") * _NC + lax.axis_index("c")
    base = wid * _BPW
    pltpu.sync_copy(idx_hbm.at[pl.ds(base, _BPW)], idx_v)
    pltpu.async_copy(tab_hbm.at[idx_v], rows_v, sem).wait()
    pltpu.sync_copy(rows_v, out_hbm.at[pl.ds(base, _BPW)])


@functools.cache
def _gather_call():
    return pl.kernel(
        _gather_body,
        mesh=plsc.VectorSubcoreMesh(core_axis_name="c", subcore_axis_name="s",
                                    num_cores=_NC, num_subcores=_NS),
        out_type=jax.ShapeDtypeStruct((B, D), jnp.float32),
        scratch_types=[
            pltpu.VMEM((_BPW,), jnp.int32),
            pltpu.VMEM((_BPW, D), jnp.float32),
            pltpu.SemaphoreType.DMA,
        ],
    )


def _mlp_body(prev_ref, W1_ref, b1_ref, Wu_ref, bu_ref, Ws_ref, bs_ref,
              mu_ref, ls_ref):
    prev = prev_ref[...]
    h = lax.dot_general(prev, W1_ref[...], (((1,), (1,)), ((), ())),
                        preferred_element_type=jnp.float32)
    h = jnp.maximum(h + b1_ref[...], 0.0)
    mu_ref[...] = lax.dot_general(h, Wu_ref[...], (((1,), (1,)), ((), ())),
                                  preferred_element_type=jnp.float32) + bu_ref[...]
    ls_ref[...] = lax.dot_general(h, Ws_ref[...], (((1,), (1,)), ((), ())),
                                  preferred_element_type=jnp.float32) + bs_ref[...]


def _mlp_call(prev, W1, b1, Wu, bu, Ws, bs):
    return pl.pallas_call(
        _mlp_body,
        out_shape=(
            jax.ShapeDtypeStruct((B, D), jnp.float32),
            jax.ShapeDtypeStruct((B, D), jnp.float32),
        ),
    )(prev, W1, b1.reshape(1, H), Wu, bu.reshape(1, D), Ws, bs.reshape(1, D))


def kernel(codes, codes_table, W1, b1, Wu, bu, Ws, bs):
    idx = _argmin_call(codes, codes_table).reshape(B)
    prev = _gather_call()(codes_table, idx)
    return _mlp_call(prev, W1, b1, Wu, bu, Ws, bs)


# speedup vs baseline: 1.0701x; 1.0701x over previous
import functools

import jax
import jax.numpy as jnp
from jax import lax
from jax.experimental import pallas as pl
from jax.experimental.pallas import tpu as pltpu
from jax.experimental.pallas import tpu_sc as plsc

B = 1024
N = 100000
D = 128
H = 512

TN = 4096
GRID = (N + TN - 1) // TN

_NC = 2
_NS = 16
_NW = _NC * _NS
_BPW = B // _NW


_RT = 64
_NS_STRIPS = TN // 128
_BIG = 3.4e38


_G = 4
_NG = _NS_STRIPS // _G
_GW = _G * 128


def _argmin_body(codes_ref, tab_ref, out_ref, runm_ref, runi_ref,
                 bufa_ref, bufb_ref, c2a_ref, c2b_ref):
    step = pl.program_id(0)

    @pl.when(step == 0)
    def _init():
        runm_ref[...] = jnp.full((B, 128), _BIG, jnp.float32)
        bufb_ref[...] = jnp.zeros((B, _GW), jnp.float32)
        c2b_ref[...] = jnp.full((1, _GW), _BIG, jnp.float32)

    codes = codes_ref[...]
    q2 = jnp.sum(codes * codes, axis=1, keepdims=True)
    codes2 = codes + codes
    tab = tab_ref[...]
    c2 = jnp.sum(tab * tab, axis=1)[None, :]
    col = lax.broadcasted_iota(jnp.int32, (1, TN), 1)
    c2 = jnp.where(col < N - step * TN, c2, _BIG)

    def produce(g, buf, c2buf):
        csl = slice(g * _GW, (g + 1) * _GW)
        c2buf[...] = c2[:, csl]
        buf[...] = lax.dot_general(
            codes2, tab_ref[csl, :], (((1,), (1,)), ((), ())),
            preferred_element_type=jnp.float32)

    def consume(gid0, buf, c2buf):
        c2g = c2buf[...]
        for rt in range(B // _RT):
            rsl = slice(rt * _RT, (rt + 1) * _RT)
            q2t = q2[rsl, :]
            rm = runm_ref[rsl, :]
            ri = runi_ref[rsl, :]
            for j in range(_G):
                jsl = slice(j * 128, (j + 1) * 128)
                d2k = (q2t + c2g[:, jsl]) - buf[rsl, jsl]
                ck = d2k < rm
                rm = jnp.where(ck, d2k, rm)
                ri = jnp.where(ck, gid0 + jnp.int32(j), ri)
            runm_ref[rsl, :] = rm
            runi_ref[rsl, :] = ri

    for g in range(_NG):
        cur, prv = ((bufa_ref, c2a_ref), (bufb_ref, c2b_ref))
        if g % 2:
            cur, prv = prv, cur
        produce(g, *cur)
        consume(step * _NS_STRIPS + (g - 1) * _G, *prv)

    @pl.when(step == pl.num_programs(0) - 1)
    def _tail():
        consume(step * _NS_STRIPS + (_NG - 1) * _G, bufb_ref, c2b_ref)
        for rt in range(B // _RT):
            rsl = slice(rt * _RT, (rt + 1) * _RT)
            rm = runm_ref[rsl, :]
            ri = runi_ref[rsl, :]
            m = jnp.min(rm, axis=1, keepdims=True)
            gi = ri * 128 + lax.broadcasted_iota(jnp.int32, (_RT, 128), 1)
            am = jnp.min(jnp.where(rm == m, gi, jnp.int32(2**31 - 1)),
                         axis=1, keepdims=True)
            out_ref[rsl, :] = am


def _argmin_call(codes, codes_table):
    return pl.pallas_call(
        _argmin_body,
        grid=(GRID,),
        in_specs=[
            pl.BlockSpec((B, D), lambda i: (0, 0)),
            pl.BlockSpec((TN, D), lambda i: (i, 0)),
        ],
        out_specs=pl.BlockSpec((B, 1), lambda i: (0, 0)),
        out_shape=jax.ShapeDtypeStruct((B, 1), jnp.int32),
        scratch_shapes=[
            pltpu.VMEM((B, 128), jnp.float32),
            pltpu.VMEM((B, 128), jnp.int32),
            pltpu.VMEM((B, _GW), jnp.float32),
            pltpu.VMEM((B, _GW), jnp.float32),
            pltpu.VMEM((1, _GW), jnp.float32),
            pltpu.VMEM((1, _GW), jnp.float32),
        ],
        compiler_params=pltpu.CompilerParams(
            dimension_semantics=("arbitrary",),
        ),
    )(codes, codes_table)


def _gather_body(tab_hbm, idx_hbm, out_hbm, idx_v, rows_v, sem):
    wid = lax.axis_index("s") * _NC + lax.axis_index("c")
    base = wid * _BPW
    pltpu.sync_copy(idx_hbm.at[pl.ds(base, _BPW)], idx_v)
    pltpu.async_copy(tab_hbm.at[idx_v], rows_v, sem).wait()
    pltpu.sync_copy(rows_v, out_hbm.at[pl.ds(base, _BPW)])


@functools.cache
def _gather_call():
    return pl.kernel(
        _gather_body,
        mesh=plsc.VectorSubcoreMesh(core_axis_name="c", subcore_axis_name="s",
                                    num_cores=_NC, num_subcores=_NS),
        out_type=jax.ShapeDtypeStruct((B, D), jnp.float32),
        scratch_types=[
            pltpu.VMEM((_BPW,), jnp.int32),
            pltpu.VMEM((_BPW, D), jnp.float32),
            pltpu.SemaphoreType.DMA,
        ],
    )


def _mlp_body(prev_ref, W1_ref, b1_ref, Wu_ref, bu_ref, Ws_ref, bs_ref,
              mu_ref, ls_ref):
    prev = prev_ref[...]
    h = lax.dot_general(prev, W1_ref[...], (((1,), (1,)), ((), ())),
                        preferred_element_type=jnp.float32)
    h = jnp.maximum(h + b1_ref[...], 0.0)
    mu_ref[...] = lax.dot_general(h, Wu_ref[...], (((1,), (1,)), ((), ())),
                                  preferred_element_type=jnp.float32) + bu_ref[...]
    ls_ref[...] = lax.dot_general(h, Ws_ref[...], (((1,), (1,)), ((), ())),
                                  preferred_element_type=jnp.float32) + bs_ref[...]


def _mlp_call(prev, W1, b1, Wu, bu, Ws, bs):
    return pl.pallas_call(
        _mlp_body,
        out_shape=(
            jax.ShapeDtypeStruct((B, D), jnp.float32),
            jax.ShapeDtypeStruct((B, D), jnp.float32),
        ),
    )(prev, W1, b1.reshape(1, H), Wu, bu.reshape(1, D), Ws, bs.reshape(1, D))


def kernel(codes, codes_table, W1, b1, Wu, bu, Ws, bs):
    idx = _argmin_call(codes, codes_table).reshape(B)
    prev = _gather_call()(codes_table, idx)
    return _mlp_call(prev, W1, b1, Wu, bu, Ws, bs)


# speedup vs baseline: 1.0795x; 1.0089x over previous
import functools

import jax
import jax.numpy as jnp
from jax import lax
from jax.experimental import pallas as pl
from jax.experimental.pallas import tpu as pltpu
from jax.experimental.pallas import tpu_sc as plsc

B = 1024
N = 100000
D = 128
H = 512

TN = 6144
GRID = (N + TN - 1) // TN

_NC = 2
_NS = 16
_NW = _NC * _NS
_BPW = B // _NW


_RT = 64
_NS_STRIPS = TN // 128
_BIG = 3.4e38


def _argmin_body(codes_ref, tab_ref, out_ref, runm_ref, runi_ref):
    step = pl.program_id(0)

    @pl.when(step == 0)
    def _init():
        runm_ref[...] = jnp.full((B, 128), _BIG, jnp.float32)

    codes = codes_ref[...]
    tab = tab_ref[...]
    q2 = jnp.sum(codes * codes, axis=1, keepdims=True)
    c2 = jnp.sum(tab * tab, axis=1)[None, :]
    col = lax.broadcasted_iota(jnp.int32, (1, TN), 1)
    c2 = jnp.where(col < N - step * TN, c2, _BIG)
    s2 = lax.dot_general(codes + codes, tab, (((1,), (1,)), ((), ())),
                         preferred_element_type=jnp.float32)
    last = step == pl.num_programs(0) - 1
    for rt in range(B // _RT):
        rsl = slice(rt * _RT, (rt + 1) * _RT)
        q2t = q2[rsl, :]
        rm = runm_ref[rsl, :]
        ri = runi_ref[rsl, :]
        for k in range(_NS_STRIPS):
            csl = slice(k * 128, (k + 1) * 128)
            d2k = (q2t + c2[:, csl]) - s2[rsl, csl]
            ck = d2k < rm
            rm = jnp.where(ck, d2k, rm)
            ri = jnp.where(ck, jnp.int32(step * _NS_STRIPS + k), ri)
        runm_ref[rsl, :] = rm
        runi_ref[rsl, :] = ri

    @pl.when(last)
    def _tail():
        for rt in range(B // _RT):
            rsl = slice(rt * _RT, (rt + 1) * _RT)
            rm = runm_ref[rsl, :]
            ri = runi_ref[rsl, :]
            m = jnp.min(rm, axis=1, keepdims=True)
            gi = ri * 128 + lax.broadcasted_iota(jnp.int32, (_RT, 128), 1)
            am = jnp.min(jnp.where(rm == m, gi, jnp.int32(2**31 - 1)),
                         axis=1, keepdims=True)
            out_ref[rsl, :] = am


def _argmin_call(codes, codes_table):
    return pl.pallas_call(
        _argmin_body,
        grid=(GRID,),
        in_specs=[
            pl.BlockSpec((B, D), lambda i: (0, 0)),
            pl.BlockSpec((TN, D), lambda i: (i, 0)),
        ],
        out_specs=pl.BlockSpec((B, 1), lambda i: (0, 0)),
        out_shape=jax.ShapeDtypeStruct((B, 1), jnp.int32),
        scratch_shapes=[
            pltpu.VMEM((B, 128), jnp.float32),
            pltpu.VMEM((B, 128), jnp.int32),
        ],
        compiler_params=pltpu.CompilerParams(
            dimension_semantics=("arbitrary",),
        ),
    )(codes, codes_table)


def _gather_body(tab_hbm, idx_hbm, out_hbm, idx_v, rows_v, sem):
    wid = lax.axis_index("s") * _NC + lax.axis_index("c")
    base = wid * _BPW
    pltpu.sync_copy(idx_hbm.at[pl.ds(base, _BPW)], idx_v)
    pltpu.async_copy(tab_hbm.at[idx_v], rows_v, sem).wait()
    pltpu.sync_copy(rows_v, out_hbm.at[pl.ds(base, _BPW)])


@functools.cache
def _gather_call():
    return pl.kernel(
        _gather_body,
        mesh=plsc.VectorSubcoreMesh(core_axis_name="c", subcore_axis_name="s",
                                    num_cores=_NC, num_subcores=_NS),
        out_type=jax.ShapeDtypeStruct((B, D), jnp.float32),
        scratch_types=[
            pltpu.VMEM((_BPW,), jnp.int32),
            pltpu.VMEM((_BPW, D), jnp.float32),
            pltpu.SemaphoreType.DMA,
        ],
    )


def _mlp_body(prev_ref, W1_ref, b1_ref, Wu_ref, bu_ref, Ws_ref, bs_ref,
              mu_ref, ls_ref):
    prev = prev_ref[...]
    h = lax.dot_general(prev, W1_ref[...], (((1,), (1,)), ((), ())),
                        preferred_element_type=jnp.float32)
    h = jnp.maximum(h + b1_ref[...], 0.0)
    mu_ref[...] = lax.dot_general(h, Wu_ref[...], (((1,), (1,)), ((), ())),
                                  preferred_element_type=jnp.float32) + bu_ref[...]
    ls_ref[...] = lax.dot_general(h, Ws_ref[...], (((1,), (1,)), ((), ())),
                                  preferred_element_type=jnp.float32) + bs_ref[...]


def _mlp_call(prev, W1, b1, Wu, bu, Ws, bs):
    return pl.pallas_call(
        _mlp_body,
        out_shape=(
            jax.ShapeDtypeStruct((B, D), jnp.float32),
            jax.ShapeDtypeStruct((B, D), jnp.float32),
        ),
    )(prev, W1, b1.reshape(1, H), Wu, bu.reshape(1, D), Ws, bs.reshape(1, D))


def kernel(codes, codes_table, W1, b1, Wu, bu, Ws, bs):
    idx = _argmin_call(codes, codes_table).reshape(B)
    prev = _gather_call()(codes_table, idx)
    return _mlp_call(prev, W1, b1, Wu, bu, Ws, bs)


# speedup vs baseline: 1.0988x; 1.0178x over previous
import functools

import jax
import jax.numpy as jnp
from jax import lax
from jax.experimental import pallas as pl
from jax.experimental.pallas import tpu as pltpu
from jax.experimental.pallas import tpu_sc as plsc

B = 1024
N = 100000
D = 128
H = 512

TN = 4096
GRID = (N + TN - 1) // TN

_NC = 2
_NS = 16
_NW = _NC * _NS
_BPW = B // _NW


_RT = 64
_NS_STRIPS = TN // 128
_BIG = 3.4e38


def _argmin_body(codes_ref, tab_ref, out_ref, runm_ref, runi_ref):
    step = pl.program_id(0)

    @pl.when(step == 0)
    def _init():
        runm_ref[...] = jnp.full((B, 128), _BIG, jnp.float32)

    codes = codes_ref[...]
    tab = tab_ref[...]
    q2 = jnp.sum(codes * codes, axis=1, keepdims=True)
    c2 = jnp.sum(tab * tab, axis=1)[None, :]
    col = lax.broadcasted_iota(jnp.int32, (1, TN), 1)
    c2 = jnp.where(col < N - step * TN, c2, _BIG)
    s2 = lax.dot_general(codes + codes, tab, (((1,), (1,)), ((), ())),
                         preferred_element_type=jnp.float32)
    last = step == pl.num_programs(0) - 1
    for rt in range(B // _RT):
        rsl = slice(rt * _RT, (rt + 1) * _RT)
        q2t = q2[rsl, :]
        rm = runm_ref[rsl, :]
        ri = runi_ref[rsl, :]
        for k in range(_NS_STRIPS):
            csl = slice(k * 128, (k + 1) * 128)
            d2k = (q2t + c2[:, csl]) - s2[rsl, csl]
            ck = d2k < rm
            rm = jnp.where(ck, d2k, rm)
            ri = jnp.where(ck, jnp.int32(step * _NS_STRIPS + k), ri)
        runm_ref[rsl, :] = rm
        runi_ref[rsl, :] = ri

    @pl.when(last)
    def _tail():
        for rt in range(B // _RT):
            rsl = slice(rt * _RT, (rt + 1) * _RT)
            rm = runm_ref[rsl, :]
            ri = runi_ref[rsl, :]
            m = jnp.min(rm, axis=1, keepdims=True)
            gi = ri * 128 + lax.broadcasted_iota(jnp.int32, (_RT, 128), 1)
            am = jnp.min(jnp.where(rm == m, gi, jnp.int32(2**31 - 1)),
                         axis=1, keepdims=True)
            out_ref[rsl, :] = am


def _argmin_call(codes, codes_table):
    return pl.pallas_call(
        _argmin_body,
        grid=(GRID,),
        in_specs=[
            pl.BlockSpec((B, D), lambda i: (0, 0)),
            pl.BlockSpec((TN, D), lambda i: (i, 0)),
        ],
        out_specs=pl.BlockSpec((B, 1), lambda i: (0, 0)),
        out_shape=jax.ShapeDtypeStruct((B, 1), jnp.int32),
        scratch_shapes=[
            pltpu.VMEM((B, 128), jnp.float32),
            pltpu.VMEM((B, 128), jnp.int32),
        ],
        compiler_params=pltpu.CompilerParams(
            dimension_semantics=("arbitrary",),
        ),
    )(codes, codes_table)


def _gather_body(tab_hbm, idx_hbm, out_hbm, idx_v, rows_v, sem):
    wid = lax.axis_index("s") * _NC + lax.axis_index("c")
    base = wid * _BPW
    pltpu.sync_copy(idx_hbm.at[pl.ds(base, _BPW)], idx_v)
    pltpu.async_copy(tab_hbm.at[idx_v], rows_v, sem).wait()
    pltpu.sync_copy(rows_v, out_hbm.at[pl.ds(base, _BPW)])


@functools.cache
def _gather_call():
    return pl.kernel(
        _gather_body,
        mesh=plsc.VectorSubcoreMesh(core_axis_name="c", subcore_axis_name="s",
                                    num_cores=_NC, num_subcores=_NS),
        out_type=jax.ShapeDtypeStruct((B, D), jnp.float32),
        scratch_types=[
            pltpu.VMEM((_BPW,), jnp.int32),
            pltpu.VMEM((_BPW, D), jnp.float32),
            pltpu.SemaphoreType.DMA,
        ],
    )


def _mlp_body(prev_ref, W1_ref, b1_ref, Wu_ref, bu_ref, Ws_ref, bs_ref,
              mu_ref, ls_ref):
    prev = prev_ref[...]
    h = lax.dot_general(prev, W1_ref[...], (((1,), (1,)), ((), ())),
                        preferred_element_type=jnp.float32)
    h = jnp.maximum(h + b1_ref[...], 0.0)
    mu_ref[...] = lax.dot_general(h, Wu_ref[...], (((1,), (1,)), ((), ())),
                                  preferred_element_type=jnp.float32) + bu_ref[...]
    ls_ref[...] = lax.dot_general(h, Ws_ref[...], (((1,), (1,)), ((), ())),
                                  preferred_element_type=jnp.float32) + bs_ref[...]


def _mlp_call(prev, W1, b1, Wu, bu, Ws, bs):
    return pl.pallas_call(
        _mlp_body,
        out_shape=(
            jax.ShapeDtypeStruct((B, D), jnp.float32),
            jax.ShapeDtypeStruct((B, D), jnp.float32),
        ),
    )(prev, W1, b1.reshape(1, H), Wu, bu.reshape(1, D), Ws, bs.reshape(1, D))


def kernel(codes, codes_table, W1, b1, Wu, bu, Ws, bs):
    idx = _argmin_call(codes, codes_table).reshape(B)
    prev = _gather_call()(codes_table, idx)
    return _mlp_call(prev, W1, b1, Wu, bu, Ws, bs)
